# baseline (speedup 1.0000x reference)
import jax
import jax.numpy as jnp
from jax import lax
from jax.experimental import pallas as pl
from jax.experimental.pallas import tpu as pltpu

B, C, N, NT, K = 256, 256, 4096, 32, 8
QB = 16


def _topk_body(q_ref, s_ref, g_ref, gm_ref, bt_ref, idx_ref, gate_ref, aff_ref):
    q = q_ref[...]
    s = s_ref[...]
    qn = q / jnp.maximum(jnp.sqrt(jnp.sum(q * q, axis=1, keepdims=True)), 1e-12)
    sn = s / jnp.maximum(jnp.sqrt(jnp.sum(s * s, axis=1, keepdims=True)), 1e-12)
    sims = lax.dot_general(
        qn.astype(jnp.bfloat16), sn.astype(jnp.bfloat16),
        dimension_numbers=(((1,), (1,)), ((), ())),
        preferred_element_type=jnp.float32,
    )
    iota = lax.broadcasted_iota(jnp.int32, (B, N), 1)
    neg = jnp.float32(-jnp.inf)
    cols = []
    for _ in range(K):
        m = jnp.max(sims, axis=1, keepdims=True)
        idxk = jnp.min(jnp.where(sims >= m, iota, N), axis=1)
        cols.append(idxk)
        sims = jnp.where(iota == idxk[:, None], neg, sims)
    idx_ref[...] = jnp.stack(cols, axis=1)
    gate = jax.nn.sigmoid(g_ref[0, 0])
    gate_ref[...] = jnp.full((B, 1), gate, jnp.float32)
    aff_ref[0, :] = gm_ref[0, :] * gate
    aff_ref[1, :] = bt_ref[0, :] * gate


def _proj_body(idx_ref, *refs):
    t_refs = refs[:QB * K]
    w_ref, b_ref, aff_ref, ones_ref, out_ref = refs[QB * K:]
    x = jnp.concatenate(
        [t[0] for t in t_refs], axis=0
    ).astype(jnp.bfloat16)
    h = lax.dot_general(
        x, w_ref[...],
        dimension_numbers=(((1,), (1,)), ((), ())),
        preferred_element_type=jnp.float32,
    ) + b_ref[...]
    hb = h.astype(jnp.bfloat16)
    g = ones_ref[...]
    mu = lax.dot_general(
        hb, g, dimension_numbers=(((1,), (0,)), ((), ())),
        preferred_element_type=jnp.float32,
    )[:, 0:1]
    m2 = lax.dot_general(
        hb * hb, g, dimension_numbers=(((1,), (0,)), ((), ())),
        preferred_element_type=jnp.float32,
    )[:, 0:1]
    r = lax.rsqrt(m2 - mu * mu + 1e-5)
    z = (hb - mu.astype(jnp.bfloat16)) * r.astype(jnp.bfloat16)
    hn = z * aff_ref[0:1, :].astype(jnp.bfloat16) + aff_ref[1:2, :].astype(jnp.bfloat16)
    out_ref[...] = hn.astype(jnp.float32).reshape(QB, K * NT, C)


def kernel(query_pre_summary, bank_summaries, bank_templates, W, b, gamma, beta, gate_logit):
    g_arr = jnp.reshape(gate_logit.astype(jnp.float32), (1, 1))
    idx, gate_b, aff = pl.pallas_call(
        _topk_body,
        out_shape=[
            jax.ShapeDtypeStruct((B, K), jnp.int32),
            jax.ShapeDtypeStruct((B, 1), jnp.float32),
            jax.ShapeDtypeStruct((2, C), jnp.float32),
        ],
    )(query_pre_summary, bank_summaries, g_arr,
      jnp.reshape(gamma, (1, C)), jnp.reshape(beta, (1, C)))

    def t_map(o, k):
        def m(bb, idx_ref):
            return (idx_ref[QB * bb + o, k], 0, 0)
        return m

    grid_spec = pltpu.PrefetchScalarGridSpec(
        num_scalar_prefetch=1,
        grid=(B // QB,),
        in_specs=(
            [pl.BlockSpec((1, NT, C), t_map(o, k))
             for o in range(QB) for k in range(K)]
            + [
                pl.BlockSpec((C, C), lambda bb, idx_ref: (0, 0)),
                pl.BlockSpec((1, C), lambda bb, idx_ref: (0, 0)),
                pl.BlockSpec((2, C), lambda bb, idx_ref: (0, 0)),
                pl.BlockSpec((C, 128), lambda bb, idx_ref: (0, 0)),
            ]
        ),
        out_specs=pl.BlockSpec((QB, K * NT, C), lambda bb, idx_ref: (bb, 0, 0)),
    )
    retrieved = pl.pallas_call(
        _proj_body,
        grid_spec=grid_spec,
        out_shape=jax.ShapeDtypeStruct((B, K * NT, C), jnp.float32),
    )(
        idx,
        *([bank_templates] * (QB * K)),
        W.astype(jnp.bfloat16),
        jnp.reshape(b, (1, C)),
        aff,
        jnp.zeros((C, 128), jnp.bfloat16).at[:, 0].set(jnp.bfloat16(1.0 / C)),
    )
    return retrieved, gate_b


# speedup vs baseline: 1.1059x; 1.1059x over previous
import jax
import jax.numpy as jnp
from jax import lax
from jax.experimental import pallas as pl
from jax.experimental.pallas import tpu as pltpu

B, C, N, NT, K = 256, 256, 4096, 32, 8
QB = 32


def _topk_body(q_ref, s_ref, g_ref, gm_ref, bt_ref, idx_ref, gate_ref, aff_ref):
    q = q_ref[...]
    s = s_ref[...]
    qn = q / jnp.maximum(jnp.sqrt(jnp.sum(q * q, axis=1, keepdims=True)), 1e-12)
    sn = s / jnp.maximum(jnp.sqrt(jnp.sum(s * s, axis=1, keepdims=True)), 1e-12)
    sims = lax.dot_general(
        qn.astype(jnp.bfloat16), sn.astype(jnp.bfloat16),
        dimension_numbers=(((1,), (1,)), ((), ())),
        preferred_element_type=jnp.float32,
    )
    iota = lax.broadcasted_iota(jnp.int32, (B, N), 1)
    neg = jnp.float32(-jnp.inf)
    cols = []
    for _ in range(K):
        m = jnp.max(sims, axis=1, keepdims=True)
        idxk = jnp.min(jnp.where(sims >= m, iota, N), axis=1)
        cols.append(idxk)
        sims = jnp.where(iota == idxk[:, None], neg, sims)
    idx_ref[...] = jnp.stack(cols, axis=1)
    gate = jax.nn.sigmoid(g_ref[0, 0])
    gate_ref[...] = jnp.full((B, 1), gate, jnp.float32)
    aff_ref[0, :] = gm_ref[0, :] * gate
    aff_ref[1, :] = bt_ref[0, :] * gate


def _proj_body(idx_ref, *refs):
    t_refs = refs[:QB * K]
    w_ref, b_ref, aff_ref, ones_ref, out_ref = refs[QB * K:]
    x = jnp.concatenate(
        [t[0] for t in t_refs], axis=0
    ).astype(jnp.bfloat16)
    h = lax.dot_general(
        x, w_ref[...],
        dimension_numbers=(((1,), (1,)), ((), ())),
        preferred_element_type=jnp.float32,
    ) + b_ref[...]
    del ones_ref
    mu = jnp.mean(h, axis=1, keepdims=True)
    m2 = jnp.mean(h * h, axis=1, keepdims=True)
    r = lax.rsqrt(m2 - mu * mu + 1e-5)
    hn = (h - mu) * r * aff_ref[0:1, :] + aff_ref[1:2, :]
    out_ref[...] = hn.reshape(QB, K * NT, C)


def kernel(query_pre_summary, bank_summaries, bank_templates, W, b, gamma, beta, gate_logit):
    g_arr = jnp.reshape(gate_logit.astype(jnp.float32), (1, 1))
    idx, gate_b, aff = pl.pallas_call(
        _topk_body,
        out_shape=[
            jax.ShapeDtypeStruct((B, K), jnp.int32),
            jax.ShapeDtypeStruct((B, 1), jnp.float32),
            jax.ShapeDtypeStruct((2, C), jnp.float32),
        ],
    )(query_pre_summary, bank_summaries, g_arr,
      jnp.reshape(gamma, (1, C)), jnp.reshape(beta, (1, C)))

    def t_map(o, k):
        def m(bb, idx_ref):
            return (idx_ref[QB * bb + o, k], 0, 0)
        return m

    grid_spec = pltpu.PrefetchScalarGridSpec(
        num_scalar_prefetch=1,
        grid=(B // QB,),
        in_specs=(
            [pl.BlockSpec((1, NT, C), t_map(o, k))
             for o in range(QB) for k in range(K)]
            + [
                pl.BlockSpec((C, C), lambda bb, idx_ref: (0, 0)),
                pl.BlockSpec((1, C), lambda bb, idx_ref: (0, 0)),
                pl.BlockSpec((2, C), lambda bb, idx_ref: (0, 0)),
                pl.BlockSpec((C, 128), lambda bb, idx_ref: (0, 0)),
            ]
        ),
        out_specs=pl.BlockSpec((QB, K * NT, C), lambda bb, idx_ref: (bb, 0, 0)),
    )
    retrieved = pl.pallas_call(
        _proj_body,
        grid_spec=grid_spec,
        out_shape=jax.ShapeDtypeStruct((B, K * NT, C), jnp.float32),
    )(
        idx,
        *([bank_templates] * (QB * K)),
        W.astype(jnp.bfloat16),
        jnp.reshape(b, (1, C)),
        aff,
        jnp.zeros((C, 128), jnp.bfloat16).at[:, 0].set(jnp.bfloat16(1.0 / C)),
    )
    return retrieved, gate_b


# speedup vs baseline: 1.1309x; 1.0226x over previous
import jax
import jax.numpy as jnp
from jax import lax
from jax.experimental import pallas as pl
from jax.experimental.pallas import tpu as pltpu

B, C, N, NT, K = 256, 256, 4096, 32, 8
QB = 16


def _topk_body(q_ref, s_ref, g_ref, gm_ref, bt_ref, idx_ref, gate_ref, aff_ref):
    q = q_ref[...]
    s = s_ref[...]
    qn = q / jnp.maximum(jnp.sqrt(jnp.sum(q * q, axis=1, keepdims=True)), 1e-12)
    sn = s / jnp.maximum(jnp.sqrt(jnp.sum(s * s, axis=1, keepdims=True)), 1e-12)
    sims = lax.dot_general(
        qn.astype(jnp.bfloat16), sn.astype(jnp.bfloat16),
        dimension_numbers=(((1,), (1,)), ((), ())),
        preferred_element_type=jnp.float32,
    )
    iota = lax.broadcasted_iota(jnp.int32, (B, N), 1)
    neg = jnp.float32(-jnp.inf)
    cols = []
    for _ in range(K):
        m = jnp.max(sims, axis=1, keepdims=True)
        idxk = jnp.min(jnp.where(sims >= m, iota, N), axis=1)
        cols.append(idxk)
        sims = jnp.where(iota == idxk[:, None], neg, sims)
    idx_ref[...] = jnp.stack(cols, axis=1)
    gate = jax.nn.sigmoid(g_ref[0, 0])
    gate_ref[...] = jnp.full((B, 1), gate, jnp.float32)
    aff_ref[0, :] = gm_ref[0, :] * gate
    aff_ref[1, :] = bt_ref[0, :] * gate


def _proj_body(idx_ref, *refs):
    t_refs = refs[:QB * K]
    w_ref, b_ref, aff_ref, out_ref = refs[QB * K:]
    x = jnp.concatenate(
        [t[0] for t in t_refs], axis=0
    ).astype(jnp.bfloat16)
    h = lax.dot_general(
        x, w_ref[...],
        dimension_numbers=(((1,), (1,)), ((), ())),
        preferred_element_type=jnp.float32,
    ) + b_ref[...]
    mu = jnp.mean(h, axis=1, keepdims=True)
    m2 = jnp.mean(h * h, axis=1, keepdims=True)
    r = lax.rsqrt(m2 - mu * mu + 1e-5)
    hn = (h - mu) * r * aff_ref[0:1, :] + aff_ref[1:2, :]
    out_ref[...] = hn.reshape(QB, K * NT, C)


def kernel(query_pre_summary, bank_summaries, bank_templates, W, b, gamma, beta, gate_logit):
    g_arr = jnp.reshape(gate_logit.astype(jnp.float32), (1, 1))
    idx, gate_b, aff = pl.pallas_call(
        _topk_body,
        out_shape=[
            jax.ShapeDtypeStruct((B, K), jnp.int32),
            jax.ShapeDtypeStruct((B, 1), jnp.float32),
            jax.ShapeDtypeStruct((2, C), jnp.float32),
        ],
    )(query_pre_summary, bank_summaries, g_arr,
      jnp.reshape(gamma, (1, C)), jnp.reshape(beta, (1, C)))

    def t_map(o, k):
        def m(bb, idx_ref):
            return (idx_ref[QB * bb + o, k], 0, 0)
        return m

    grid_spec = pltpu.PrefetchScalarGridSpec(
        num_scalar_prefetch=1,
        grid=(B // QB,),
        in_specs=(
            [pl.BlockSpec((1, NT, C), t_map(o, k))
             for o in range(QB) for k in range(K)]
            + [
                pl.BlockSpec((C, C), lambda bb, idx_ref: (0, 0)),
                pl.BlockSpec((1, C), lambda bb, idx_ref: (0, 0)),
                pl.BlockSpec((2, C), lambda bb, idx_ref: (0, 0)),
            ]
        ),
        out_specs=pl.BlockSpec((QB, K * NT, C), lambda bb, idx_ref: (bb, 0, 0)),
    )
    retrieved = pl.pallas_call(
        _proj_body,
        grid_spec=grid_spec,
        out_shape=jax.ShapeDtypeStruct((B, K * NT, C), jnp.float32),
    )(
        idx,
        *([bank_templates] * (QB * K)),
        W.astype(jnp.bfloat16),
        jnp.reshape(b, (1, C)),
        aff,
    )
    return retrieved, gate_b
